# baseline (speedup 1.0000x reference)
import functools

import jax
import jax.numpy as jnp
from jax import lax
from jax.experimental import pallas as pl
from jax.experimental.pallas import tpu as pltpu
from jax.experimental.pallas import tpu_sc as plsc

_N_IDX = 16384
_D = 64
_LANES = 16
_NC = 2
_NS = 16
_NW = _NC * _NS
_BPW = _N_IDX // _NW
_CHUNK = 128
_NCHUNK = _BPW // _CHUNK


def _bow_partials(words2d, table):
    mesh = plsc.VectorSubcoreMesh(core_axis_name="c", subcore_axis_name="s")

    @functools.partial(
        pl.kernel,
        out_type=jax.ShapeDtypeStruct((_NW, _D), jnp.float32),
        mesh=mesh,
        scratch_types=[
            pltpu.VMEM((_NCHUNK, _CHUNK), jnp.int32),
            pltpu.VMEM((_BPW, _D), jnp.float32),
            pltpu.VMEM((_D,), jnp.float32),
            pltpu.SemaphoreType.DMA,
        ],
        compiler_params=pltpu.CompilerParams(use_tc_tiling_on_sc=False),
    )
    def body(words_hbm, table_hbm, out_hbm, idx_v, rows_v, part_v, sem):
        wid = lax.axis_index("s") * _NC + lax.axis_index("c")
        pltpu.sync_copy(words_hbm.at[wid], idx_v)
        copies = [
            pltpu.async_copy(
                table_hbm.at[idx_v.at[j]],
                rows_v.at[pl.ds(j * _CHUNK, _CHUNK)],
                sem,
            )
            for j in range(_NCHUNK)
        ]
        for c in copies:
            c.wait()

        nacc = _D // _LANES

        def red(i, acc):
            return tuple(
                acc[j] + rows_v[i, pl.ds(j * _LANES, _LANES)]
                for j in range(nacc)
            )

        acc = lax.fori_loop(
            0, _BPW, red,
            tuple(jnp.zeros((_LANES,), jnp.float32) for _ in range(nacc)),
            unroll=4,
        )
        for j in range(nacc):
            part_v[pl.ds(j * _LANES, _LANES)] = acc[j]
        pltpu.sync_copy(part_v, out_hbm.at[wid])

    return body(words2d, table)


def _combine(partials, bias2d):
    def body(p_ref, b_ref, o_ref):
        o_ref[...] = jnp.sum(p_ref[...], axis=0, keepdims=True) + b_ref[...]

    return pl.pallas_call(
        body,
        out_shape=jax.ShapeDtypeStruct((1, _D), jnp.float32),
    )(partials, bias2d)


def kernel(words, embedding, bias):
    words2d = words.astype(jnp.int32).reshape(_NW, _NCHUNK, _CHUNK)
    partials = _bow_partials(words2d, embedding)
    return _combine(partials, bias.reshape(1, _D))
